# baseline (speedup 1.0000x reference)
import jax
import jax.numpy as jnp
from jax.experimental import pallas as pl

_BLK = 1024
_INTERPRET = False


def _proj_kernel(nf_ref, w_ref, b_ref, out_ref):
    out_ref[...] = (
        jnp.dot(nf_ref[...], w_ref[...], preferred_element_type=jnp.float32)
        + b_ref[...]
    )


def _level_kernel(xiou_ref, xfp_ref, hf_ref, hs_ref, mf_ref, cmf_ref,
                  uiou_ref, uf_ref, hs_out_ref, hfch_ref):
    h = xfp_ref.shape[1]
    hsum = hf_ref[:, 0:h]
    fc = hf_ref[:, h:2 * h]
    iou = xiou_ref[...] + jnp.dot(hsum, uiou_ref[...],
                                  preferred_element_type=jnp.float32)
    i = jax.nn.sigmoid(iou[:, 0:h])
    o = jax.nn.sigmoid(iou[:, h:2 * h])
    u = jnp.tanh(iou[:, 2 * h:3 * h])
    c = i * u + fc
    hv = o * jnp.tanh(c)
    f = jax.nn.sigmoid(xfp_ref[...] + jnp.dot(hv, uf_ref[...],
                                              preferred_element_type=jnp.float32))
    m = mf_ref[...]
    cm = cmf_ref[...]
    hs_out_ref[...] = hs_ref[...] * (1.0 - m) + hv * m
    hfch_ref[:, 0:h] = hv * cm
    hfch_ref[:, h:2 * h] = (f * c) * cm


def kernel(node_features, parent_raw, W_iou, U_iou_w, U_iou_b, W_f, U_f_w, U_f_b):
    N, D = node_features.shape
    H = U_f_b.shape[0]
    B = _BLK
    Np = ((N + B - 1) // B) * B
    nb = Np // B
    f32 = jnp.float32

    ar = jnp.arange(N, dtype=jnp.int32)
    raw = parent_raw.astype(jnp.int32)
    parent = jnp.where(ar == 0, -1, raw % jnp.maximum(ar, 1))

    def dbl_cond(carry):
        anc, _ = carry
        return jnp.any(anc >= 0)

    def dbl(carry):
        anc, dep = carry
        a = jnp.maximum(anc, 0)
        g = jnp.stack([anc, dep], axis=1)[a]
        live = anc >= 0
        dep = dep + jnp.where(live, g[:, 1], 0)
        anc = jnp.where(live, g[:, 0], -1)
        return anc, dep

    _, dep = jax.lax.while_loop(
        dbl_cond, dbl, (parent, (parent >= 0).astype(jnp.int32)))
    maxd = jnp.max(dep)

    Wcat_t = jnp.concatenate([W_iou, W_f], axis=0).T
    bcat = jnp.concatenate([U_iou_b, jnp.zeros((H,), f32)]).reshape(1, 4 * H)
    nf_p = jnp.zeros((Np, D), f32).at[:N].set(node_features)

    X = pl.pallas_call(
        _proj_kernel,
        grid=(nb,),
        in_specs=[
            pl.BlockSpec((B, D), lambda i: (i, 0)),
            pl.BlockSpec((D, 4 * H), lambda i: (0, 0)),
            pl.BlockSpec((1, 4 * H), lambda i: (0, 0)),
        ],
        out_specs=pl.BlockSpec((B, 4 * H), lambda i: (i, 0)),
        out_shape=jax.ShapeDtypeStruct((Np, 4 * H), f32),
        interpret=_INTERPRET,
    )(nf_p, Wcat_t, bcat)

    Xiou = X[:, :3 * H]
    P = X[:, 3 * H:]
    pclamp = jnp.maximum(parent, 0)
    XfP = jnp.zeros((Np, H), f32).at[:N].set(P[pclamp] + U_f_b)

    dep_p = jnp.full((Np,), -1, jnp.int32).at[:N].set(dep)
    parent_p = jnp.full((Np,), -1, jnp.int32).at[:N].set(parent)
    rows = jnp.arange(Np, dtype=jnp.int32)

    UiouT = U_iou_w.T
    UfT = U_f_w.T

    level_call = pl.pallas_call(
        _level_kernel,
        grid=(nb,),
        in_specs=[
            pl.BlockSpec((B, 3 * H), lambda i: (i, 0)),
            pl.BlockSpec((B, H), lambda i: (i, 0)),
            pl.BlockSpec((B, 2 * H), lambda i: (i, 0)),
            pl.BlockSpec((B, H), lambda i: (i, 0)),
            pl.BlockSpec((B, 1), lambda i: (i, 0)),
            pl.BlockSpec((B, 1), lambda i: (i, 0)),
            pl.BlockSpec((H, 3 * H), lambda i: (0, 0)),
            pl.BlockSpec((H, H), lambda i: (0, 0)),
        ],
        out_specs=[
            pl.BlockSpec((B, H), lambda i: (i, 0)),
            pl.BlockSpec((B, 2 * H), lambda i: (i, 0)),
        ],
        out_shape=[
            jax.ShapeDtypeStruct((Np, H), f32),
            jax.ShapeDtypeStruct((Np, 2 * H), f32),
        ],
        input_output_aliases={3: 0},
        interpret=_INTERPRET,
    )

    def body(t, carry):
        Hs, HF = carry
        L = maxd - t
        on_level = dep_p == L
        cmb = on_level & (parent_p >= 0)
        mf = on_level.astype(f32).reshape(Np, 1)
        cmf = cmb.astype(f32).reshape(Np, 1)
        Hs, hfch = level_call(Xiou, XfP, HF, Hs, mf, cmf, UiouT, UfT)
        pi = jnp.where(cmb, parent_p, rows)
        HF = HF.at[pi].add(hfch)
        return Hs, HF

    Hs, _ = jax.lax.fori_loop(
        0, maxd + 1, body,
        (jnp.zeros((Np, H), f32), jnp.zeros((Np, 2 * H), f32)))
    return Hs[:N]


# speedup vs baseline: 6.5034x; 6.5034x over previous
import jax
import jax.numpy as jnp
from jax import lax
from jax.experimental import pallas as pl
from jax.experimental.pallas import tpu as pltpu

_BLK = 1024
_CHUNK = 8192
_INTERPRET = False


def _proj_kernel(nf_ref, w_ref, b_ref, out_ref):
    out_ref[...] = (
        jnp.dot(nf_ref[...], w_ref[...], preferred_element_type=jnp.float32)
        + b_ref[...]
    )


def _level_kernel(s_ref, xiou_ref, xfp_ref, hf_ref, hs_ref, mf_ref, cmf_ref,
                  uiou_ref, uf_ref, hs_out_ref, hfch_ref):
    h = xfp_ref.shape[1]
    hsum = hf_ref[:, 0:h]
    fc = hf_ref[:, h:2 * h]
    iou = xiou_ref[...] + jnp.dot(hsum, uiou_ref[...],
                                  preferred_element_type=jnp.float32)
    i = jax.nn.sigmoid(iou[:, 0:h])
    o = jax.nn.sigmoid(iou[:, h:2 * h])
    u = jnp.tanh(iou[:, 2 * h:3 * h])
    c = i * u + fc
    hv = o * jnp.tanh(c)
    f = jax.nn.sigmoid(xfp_ref[...] + jnp.dot(hv, uf_ref[...],
                                              preferred_element_type=jnp.float32))
    m = mf_ref[...]
    cm = cmf_ref[...]
    hs_out_ref[...] = hs_ref[...] * (1.0 - m) + hv * m
    hfch_ref[:, 0:h] = hv * cm
    hfch_ref[:, h:2 * h] = (f * c) * cm


def kernel(node_features, parent_raw, W_iou, U_iou_w, U_iou_b, W_f, U_f_w, U_f_b):
    N, D = node_features.shape
    H = U_f_b.shape[0]
    B = _BLK if N >= _BLK else 64
    Np = ((N + B - 1) // B) * B
    C = min(_CHUNK, Np)
    f32 = jnp.float32

    ar = jnp.arange(N, dtype=jnp.int32)
    raw = parent_raw.astype(jnp.int32)
    parent = jnp.where(ar == 0, -1, raw % jnp.maximum(ar, 1))

    def dbl_cond(carry):
        anc, _ = carry
        return jnp.any(anc >= 0)

    def dbl(carry):
        anc, dep = carry
        a = jnp.maximum(anc, 0)
        g = jnp.stack([anc, dep], axis=1)[a]
        live = anc >= 0
        dep = dep + jnp.where(live, g[:, 1], 0)
        anc = jnp.where(live, g[:, 0], -1)
        return anc, dep

    _, dep = lax.while_loop(
        dbl_cond, dbl, (parent, (parent >= 0).astype(jnp.int32)))
    maxd = jnp.max(dep)

    dep_s, ord_ = lax.sort_key_val(dep, ar)
    rank = jnp.zeros((N,), jnp.int32).at[ord_].set(ar)
    parent_ord = parent[ord_]
    pvalid = parent_ord >= 0
    pps = rank[jnp.maximum(parent_ord, 0)]
    nf_s = node_features[ord_]

    Wcat_t = jnp.concatenate([W_iou, W_f], axis=0).T
    bcat = jnp.concatenate([U_iou_b, jnp.zeros((H,), f32)]).reshape(1, 4 * H)
    nf_p = jnp.zeros((Np, D), f32).at[:N].set(nf_s)

    X = pl.pallas_call(
        _proj_kernel,
        grid=(Np // B,),
        in_specs=[
            pl.BlockSpec((B, D), lambda i: (i, 0)),
            pl.BlockSpec((D, 4 * H), lambda i: (0, 0)),
            pl.BlockSpec((1, 4 * H), lambda i: (0, 0)),
        ],
        out_specs=pl.BlockSpec((B, 4 * H), lambda i: (i, 0)),
        out_shape=jax.ShapeDtypeStruct((Np, 4 * H), f32),
        interpret=_INTERPRET,
    )(nf_p, Wcat_t, bcat)

    Xiou = X[:, :3 * H]
    P = X[:, 3 * H:]
    XfP = jnp.zeros((Np, H), f32).at[:N].set(P[pps] + U_f_b)

    dep_pad = jnp.full((Np,), -1, jnp.int32).at[:N].set(dep_s)
    pv_pad = jnp.zeros((Np,), jnp.bool_).at[:N].set(pvalid)
    pps_pad = jnp.zeros((Np,), jnp.int32).at[:N].set(pps)

    UiouT = U_iou_w.T
    UfT = U_f_w.T

    def _band(i, s):
        return (s[0] + i, 0)

    def _chunk(i, s):
        return (i, 0)

    level_call = pl.pallas_call(
        _level_kernel,
        grid_spec=pltpu.PrefetchScalarGridSpec(
            num_scalar_prefetch=1,
            grid=(C // B,),
            in_specs=[
                pl.BlockSpec((B, 3 * H), _band),
                pl.BlockSpec((B, H), _band),
                pl.BlockSpec((B, 2 * H), _band),
                pl.BlockSpec((B, H), _band),
                pl.BlockSpec((B, 1), _chunk),
                pl.BlockSpec((B, 1), _chunk),
                pl.BlockSpec((H, 3 * H), lambda i, s: (0, 0)),
                pl.BlockSpec((H, H), lambda i, s: (0, 0)),
            ],
            out_specs=[
                pl.BlockSpec((B, H), _band),
                pl.BlockSpec((B, 2 * H), _chunk),
            ],
        ),
        out_shape=[
            jax.ShapeDtypeStruct((Np, H), f32),
            jax.ShapeDtypeStruct((C, 2 * H), f32),
        ],
        input_output_aliases={4: 0},
        interpret=_INTERPRET,
    )

    def body(t, carry):
        Hs, HF = carry
        L = maxd - t
        lo = jnp.searchsorted(dep_s, L).astype(jnp.int32)
        hi = jnp.searchsorted(dep_s, L, side="right").astype(jnp.int32)
        base = (lo // B) * B
        nchunks = (hi - base + C - 1) // C

        def chunk_body(st):
            k, Hs, HF = st
            q0 = jnp.minimum(base + k * C, Np - C)
            dep_c = lax.dynamic_slice(dep_pad, (q0,), (C,))
            pv_c = lax.dynamic_slice(pv_pad, (q0,), (C,))
            pps_c = lax.dynamic_slice(pps_pad, (q0,), (C,))
            on = dep_c == L
            cmb = on & pv_c
            mf = on.astype(f32).reshape(C, 1)
            cmf = cmb.astype(f32).reshape(C, 1)
            sblk = (q0 // B).reshape(1)
            Hs, hfch = level_call(sblk, Xiou, XfP, HF, Hs, mf, cmf, UiouT, UfT)
            pi = jnp.where(cmb, pps_c, Np)
            HF = HF.at[pi].add(hfch, mode="drop")
            return k + 1, Hs, HF

        _, Hs, HF = lax.while_loop(
            lambda st: st[0] < nchunks, chunk_body,
            (jnp.int32(0), Hs, HF))
        return Hs, HF

    Hs, _ = lax.fori_loop(
        0, maxd + 1, body,
        (jnp.zeros((Np, H), f32), jnp.zeros((Np, 2 * H), f32)))
    return Hs[rank]
